# Initial kernel scaffold; baseline (speedup 1.0000x reference)
import jax
import jax.numpy as jnp
from jax.experimental import pallas as pl


def kernel(c_vt, domains):
    raise NotImplementedError("write your pallas kernel here")


# speedup vs baseline: 3.9845x; 3.9845x over previous
import jax
import jax.numpy as jnp
from jax.experimental import pallas as pl
from jax.experimental.pallas import tpu as pltpu

_K = 7
_KP = 8
_D = 512
_B = 16384
_R = 512
_NB = _B // _R
_MIX = 0.3


def _seg_kernel(dom_ref, x_ref, cent_ref, cntrow_ref, cntcol):
    i = pl.program_id(0)
    nb = pl.num_programs(0)
    x = x_ref[...]
    dom = dom_ref[0]
    k8 = jax.lax.broadcasted_iota(jnp.int32, (_KP, 1), 0)
    oh = (dom == k8).astype(jnp.float32)
    psum = jax.lax.dot_general(
        oh, x, (((1,), (0,)), ((), ())), preferred_element_type=jnp.float32)
    pcntc = jnp.sum(oh, axis=1, keepdims=True)
    ones_r = jnp.ones((1, oh.shape[1]), jnp.float32)
    pcntr = jax.lax.dot_general(
        ones_r, oh, (((1,), (1,)), ((), ())),
        preferred_element_type=jnp.float32)

    @pl.when(i == 0)
    def _():
        cent_ref[...] = psum
        cntrow_ref[...] = pcntr
        cntcol[...] = pcntc

    @pl.when(i > 0)
    def _():
        cent_ref[...] += psum
        cntrow_ref[...] += pcntr
        cntcol[...] += pcntc

    @pl.when(i == nb - 1)
    def _():
        cnt = cntcol[...]
        cent_ref[...] = jnp.where(
            cnt > 0.0, cent_ref[...] / jnp.maximum(cnt, 1.0), 0.0)


def _apply_kernel(cent_ref, cntrow_ref, x_ref, out_ref):
    x = x_ref[...]
    cent = cent_ref[...]
    cntrow = cntrow_ref[...]
    a2 = jnp.sum(x * x, axis=1, keepdims=True)
    cc = cent * cent
    ones_d = jnp.ones((1, _D), jnp.float32)
    b2 = jax.lax.dot_general(
        ones_d, cc, (((1,), (1,)), ((), ())),
        preferred_element_type=jnp.float32)
    xc = jax.lax.dot_general(
        x, cent, (((1,), (1,)), ((), ())),
        preferred_element_type=jnp.float32)
    d2 = jnp.maximum(a2 + b2 - 2.0 * xc, 0.0)
    dist = jnp.sqrt(d2)
    dist = jnp.where(cntrow > 0.0, dist, -1.0)
    maxd = jnp.max(dist, axis=1, keepdims=True)
    lane = jax.lax.broadcasted_iota(jnp.int32, (x.shape[0], _KP), 1)
    cand = jnp.where(dist == maxd, lane, _KP)
    idx = jnp.min(cand, axis=1, keepdims=True)
    ohsel = (lane == idx).astype(jnp.float32)
    hardest = jax.lax.dot_general(
        ohsel, cent, (((1,), (0,)), ((), ())),
        preferred_element_type=jnp.float32)
    out_ref[...] = x + _MIX * (hardest - x)


@jax.jit
def kernel(c_vt, domains):
    dom3 = domains.reshape(_NB, 1, _R)
    cent, cntrow = pl.pallas_call(
        _seg_kernel,
        grid=(_NB,),
        in_specs=[
            pl.BlockSpec((1, 1, _R), lambda i: (i, 0, 0)),
            pl.BlockSpec((_R, _D), lambda i: (i, 0)),
        ],
        out_specs=[
            pl.BlockSpec((_KP, _D), lambda i: (0, 0)),
            pl.BlockSpec((1, _KP), lambda i: (0, 0)),
        ],
        out_shape=[
            jax.ShapeDtypeStruct((_KP, _D), jnp.float32),
            jax.ShapeDtypeStruct((1, _KP), jnp.float32),
        ],
        scratch_shapes=[pltpu.VMEM((_KP, 1), jnp.float32)],
    )(dom3, c_vt)

    out = pl.pallas_call(
        _apply_kernel,
        grid=(_NB,),
        in_specs=[
            pl.BlockSpec((_KP, _D), lambda i: (0, 0)),
            pl.BlockSpec((1, _KP), lambda i: (0, 0)),
            pl.BlockSpec((_R, _D), lambda i: (i, 0)),
        ],
        out_specs=pl.BlockSpec((_R, _D), lambda i: (i, 0)),
        out_shape=jax.ShapeDtypeStruct((_B, _D), jnp.float32),
    )(cent, cntrow, c_vt)
    return out


# speedup vs baseline: 4.3192x; 1.0840x over previous
import jax
import jax.numpy as jnp
from jax.experimental import pallas as pl
from jax.experimental.pallas import tpu as pltpu

_K = 7
_KP = 8
_D = 512
_B = 16384
_R = 512
_NB = _B // _R
_MIX = 0.3
_KEEP = 1.0 - _MIX
_NEG = -1.0e30


def _seg_kernel(dom_ref, x_ref, cent_ref, cent3_ref, b2m_ref, cntcol):
    i = pl.program_id(0)
    nb = pl.num_programs(0)
    x = x_ref[...]
    dom = dom_ref[0]
    k8 = jax.lax.broadcasted_iota(jnp.int32, (_KP, 1), 0)
    oh = (dom == k8).astype(jnp.float32)
    psum = jax.lax.dot_general(
        oh, x, (((1,), (0,)), ((), ())), preferred_element_type=jnp.float32)
    pcnt = jnp.sum(oh, axis=1, keepdims=True)

    @pl.when(i == 0)
    def _():
        cent_ref[...] = psum
        cntcol[...] = pcnt

    @pl.when(i > 0)
    def _():
        cent_ref[...] += psum
        cntcol[...] += pcnt

    @pl.when(i == nb - 1)
    def _():
        cnt = cntcol[...]
        cent = jnp.where(
            cnt > 0.0, cent_ref[...] / jnp.maximum(cnt, 1.0), 0.0)
        cent_ref[...] = cent
        cent3_ref[...] = _MIX * cent
        b2 = jnp.sum(cent * cent, axis=1, keepdims=True)
        b2m_ref[...] = jnp.where(cnt > 0.0, b2, _NEG)


def _apply_kernel(cent_ref, cent3_ref, b2m_ref, x_ref, out_ref):
    x = x_ref[...]
    cent = cent_ref[...]
    xcT = jax.lax.dot_general(
        cent, x, (((1,), (1,)), ((), ())),
        preferred_element_type=jnp.float32)
    score = b2m_ref[...] - 2.0 * xcT
    m4 = jnp.maximum(score[0:4], score[4:8])
    m2 = jnp.maximum(m4[0:2], m4[2:4])
    m1 = jnp.maximum(m2[0:1], m2[1:2])
    k8 = jax.lax.broadcasted_iota(jnp.int32, (_KP, 1), 0)
    kk = jnp.where(score == m1, k8, _KP)
    i4 = jnp.minimum(kk[0:4], kk[4:8])
    i2 = jnp.minimum(i4[0:2], i4[2:4])
    idx = jnp.minimum(i2[0:1], i2[1:2])
    ohsel = (k8 == idx).astype(jnp.float32)
    hardest3 = jax.lax.dot_general(
        ohsel, cent3_ref[...], (((0,), (0,)), ((), ())),
        preferred_element_type=jnp.float32)
    out_ref[...] = _KEEP * x + hardest3


@jax.jit
def kernel(c_vt, domains):
    dom3 = domains.reshape(_NB, 1, _R)
    cent, cent3, b2m = pl.pallas_call(
        _seg_kernel,
        grid=(_NB,),
        in_specs=[
            pl.BlockSpec((1, 1, _R), lambda i: (i, 0, 0)),
            pl.BlockSpec((_R, _D), lambda i: (i, 0)),
        ],
        out_specs=[
            pl.BlockSpec((_KP, _D), lambda i: (0, 0)),
            pl.BlockSpec((_KP, _D), lambda i: (0, 0)),
            pl.BlockSpec((_KP, 1), lambda i: (0, 0)),
        ],
        out_shape=[
            jax.ShapeDtypeStruct((_KP, _D), jnp.float32),
            jax.ShapeDtypeStruct((_KP, _D), jnp.float32),
            jax.ShapeDtypeStruct((_KP, 1), jnp.float32),
        ],
        scratch_shapes=[pltpu.VMEM((_KP, 1), jnp.float32)],
    )(dom3, c_vt)

    out = pl.pallas_call(
        _apply_kernel,
        grid=(_NB,),
        in_specs=[
            pl.BlockSpec((_KP, _D), lambda i: (0, 0)),
            pl.BlockSpec((_KP, _D), lambda i: (0, 0)),
            pl.BlockSpec((_KP, 1), lambda i: (0, 0)),
            pl.BlockSpec((_R, _D), lambda i: (i, 0)),
        ],
        out_specs=pl.BlockSpec((_R, _D), lambda i: (i, 0)),
        out_shape=jax.ShapeDtypeStruct((_B, _D), jnp.float32),
    )(cent, cent3, b2m, c_vt)
    return out


# speedup vs baseline: 6.2676x; 1.4511x over previous
import jax
import jax.numpy as jnp
from jax.experimental import pallas as pl
from jax.experimental.pallas import tpu as pltpu

_K = 7
_KP = 8
_D = 512
_B = 16384
_R = 1024
_NB = _B // _R
_MIX = 0.3
_KEEP = 1.0 - _MIX
_NEG = -1.0e30


def _seg_kernel(dom_ref, x_ref, cent_ref, cent3_ref, b2m_ref, cntcol):
    i = pl.program_id(0)
    nb = pl.num_programs(0)
    x = x_ref[...]
    dom = dom_ref[0]
    k8 = jax.lax.broadcasted_iota(jnp.int32, (_KP, 1), 0)
    oh = (dom == k8).astype(jnp.float32)
    psum = jax.lax.dot_general(
        oh, x, (((1,), (0,)), ((), ())), preferred_element_type=jnp.float32)
    pcnt = jnp.sum(oh, axis=1, keepdims=True)

    @pl.when(i == 0)
    def _():
        cent_ref[...] = psum
        cntcol[...] = pcnt

    @pl.when(i > 0)
    def _():
        cent_ref[...] += psum
        cntcol[...] += pcnt

    @pl.when(i == nb - 1)
    def _():
        cnt = cntcol[...]
        cent = jnp.where(
            cnt > 0.0, cent_ref[...] / jnp.maximum(cnt, 1.0), 0.0)
        cent_ref[...] = cent
        cent3_ref[...] = _MIX * cent
        b2 = jnp.sum(cent * cent, axis=1, keepdims=True)
        b2m_ref[...] = jnp.where(cnt > 0.0, b2, _NEG)


def _apply_kernel(cent_ref, cent3_ref, b2m_ref, x_ref, out_ref):
    x = x_ref[...]
    cent = cent_ref[...]
    xcT = jax.lax.dot_general(
        cent, x, (((1,), (1,)), ((), ())),
        preferred_element_type=jnp.float32)
    score = b2m_ref[...] - 2.0 * xcT
    m4 = jnp.maximum(score[0:4], score[4:8])
    m2 = jnp.maximum(m4[0:2], m4[2:4])
    m1 = jnp.maximum(m2[0:1], m2[1:2])
    k8 = jax.lax.broadcasted_iota(jnp.int32, (_KP, 1), 0)
    kk = jnp.where(score == m1, k8, _KP)
    i4 = jnp.minimum(kk[0:4], kk[4:8])
    i2 = jnp.minimum(i4[0:2], i4[2:4])
    idx = jnp.minimum(i2[0:1], i2[1:2])
    ohsel = (k8 == idx).astype(jnp.float32)
    hardest3 = jax.lax.dot_general(
        ohsel, cent3_ref[...], (((0,), (0,)), ((), ())),
        preferred_element_type=jnp.float32)
    out_ref[...] = _KEEP * x + hardest3


@jax.jit
def kernel(c_vt, domains):
    dom3 = domains.reshape(_NB, 1, _R)
    cent, cent3, b2m = pl.pallas_call(
        _seg_kernel,
        grid=(_NB,),
        in_specs=[
            pl.BlockSpec((1, 1, _R), lambda i: (i, 0, 0)),
            pl.BlockSpec((_R, _D), lambda i: (i, 0)),
        ],
        out_specs=[
            pl.BlockSpec((_KP, _D), lambda i: (0, 0)),
            pl.BlockSpec((_KP, _D), lambda i: (0, 0)),
            pl.BlockSpec((_KP, 1), lambda i: (0, 0)),
        ],
        out_shape=[
            jax.ShapeDtypeStruct((_KP, _D), jnp.float32),
            jax.ShapeDtypeStruct((_KP, _D), jnp.float32),
            jax.ShapeDtypeStruct((_KP, 1), jnp.float32),
        ],
        scratch_shapes=[pltpu.VMEM((_KP, 1), jnp.float32)],
    )(dom3, c_vt)

    out = pl.pallas_call(
        _apply_kernel,
        grid=(_NB,),
        in_specs=[
            pl.BlockSpec((_KP, _D), lambda i: (0, 0)),
            pl.BlockSpec((_KP, _D), lambda i: (0, 0)),
            pl.BlockSpec((_KP, 1), lambda i: (0, 0)),
            pl.BlockSpec((_R, _D), lambda i: (i, 0)),
        ],
        out_specs=pl.BlockSpec((_R, _D), lambda i: (i, 0)),
        out_shape=jax.ShapeDtypeStruct((_B, _D), jnp.float32),
    )(cent, cent3, b2m, c_vt)
    return out


# speedup vs baseline: 7.5822x; 1.2098x over previous
import jax
import jax.numpy as jnp
from jax.experimental import pallas as pl
from jax.experimental.pallas import tpu as pltpu

_K = 7
_KP = 8
_D = 512
_B = 16384
_R = 2048
_NB = _B // _R
_MIX = 0.3
_KEEP = 1.0 - _MIX
_NEG = -1.0e30


def _seg_kernel(dom_ref, x_ref, cent_ref, cent3_ref, b2m_ref, cntcol):
    i = pl.program_id(0)
    nb = pl.num_programs(0)
    x = x_ref[...]
    dom = dom_ref[0]
    k8 = jax.lax.broadcasted_iota(jnp.int32, (_KP, 1), 0)
    oh = (dom == k8).astype(jnp.float32)
    psum = jax.lax.dot_general(
        oh, x, (((1,), (0,)), ((), ())), preferred_element_type=jnp.float32)
    pcnt = jnp.sum(oh, axis=1, keepdims=True)

    @pl.when(i == 0)
    def _():
        cent_ref[...] = psum
        cntcol[...] = pcnt

    @pl.when(i > 0)
    def _():
        cent_ref[...] += psum
        cntcol[...] += pcnt

    @pl.when(i == nb - 1)
    def _():
        cnt = cntcol[...]
        cent = jnp.where(
            cnt > 0.0, cent_ref[...] / jnp.maximum(cnt, 1.0), 0.0)
        cent_ref[...] = cent
        cent3_ref[...] = _MIX * cent
        b2 = jnp.sum(cent * cent, axis=1, keepdims=True)
        b2m_ref[...] = jnp.where(cnt > 0.0, b2, _NEG)


def _apply_kernel(cent_ref, cent3_ref, b2m_ref, x_ref, out_ref):
    x = x_ref[...]
    cent = cent_ref[...]
    xcT = jax.lax.dot_general(
        cent, x, (((1,), (1,)), ((), ())),
        preferred_element_type=jnp.float32)
    score = b2m_ref[...] - 2.0 * xcT
    m4 = jnp.maximum(score[0:4], score[4:8])
    m2 = jnp.maximum(m4[0:2], m4[2:4])
    m1 = jnp.maximum(m2[0:1], m2[1:2])
    k8 = jax.lax.broadcasted_iota(jnp.int32, (_KP, 1), 0)
    kk = jnp.where(score == m1, k8, _KP)
    i4 = jnp.minimum(kk[0:4], kk[4:8])
    i2 = jnp.minimum(i4[0:2], i4[2:4])
    idx = jnp.minimum(i2[0:1], i2[1:2])
    ohsel = (k8 == idx).astype(jnp.float32)
    hardest3 = jax.lax.dot_general(
        ohsel, cent3_ref[...], (((0,), (0,)), ((), ())),
        preferred_element_type=jnp.float32)
    out_ref[...] = _KEEP * x + hardest3


@jax.jit
def kernel(c_vt, domains):
    dom3 = domains.reshape(_NB, 1, _R)
    cent, cent3, b2m = pl.pallas_call(
        _seg_kernel,
        grid=(_NB,),
        in_specs=[
            pl.BlockSpec((1, 1, _R), lambda i: (i, 0, 0)),
            pl.BlockSpec((_R, _D), lambda i: (i, 0)),
        ],
        out_specs=[
            pl.BlockSpec((_KP, _D), lambda i: (0, 0)),
            pl.BlockSpec((_KP, _D), lambda i: (0, 0)),
            pl.BlockSpec((_KP, 1), lambda i: (0, 0)),
        ],
        out_shape=[
            jax.ShapeDtypeStruct((_KP, _D), jnp.float32),
            jax.ShapeDtypeStruct((_KP, _D), jnp.float32),
            jax.ShapeDtypeStruct((_KP, 1), jnp.float32),
        ],
        scratch_shapes=[pltpu.VMEM((_KP, 1), jnp.float32)],
    )(dom3, c_vt)

    out = pl.pallas_call(
        _apply_kernel,
        grid=(_NB,),
        in_specs=[
            pl.BlockSpec((_KP, _D), lambda i: (0, 0)),
            pl.BlockSpec((_KP, _D), lambda i: (0, 0)),
            pl.BlockSpec((_KP, 1), lambda i: (0, 0)),
            pl.BlockSpec((_R, _D), lambda i: (i, 0)),
        ],
        out_specs=pl.BlockSpec((_R, _D), lambda i: (i, 0)),
        out_shape=jax.ShapeDtypeStruct((_B, _D), jnp.float32),
    )(cent, cent3, b2m, c_vt)
    return out


# speedup vs baseline: 7.8745x; 1.0385x over previous
import jax
import jax.numpy as jnp
from jax.experimental import pallas as pl
from jax.experimental.pallas import tpu as pltpu

_K = 7
_KP = 8
_D = 512
_B = 16384
_R = 4096
_NB = _B // _R
_MIX = 0.3
_KEEP = 1.0 - _MIX
_NEG = -1.0e30


def _seg_kernel(dom_ref, x_ref, cent_ref, cent3_ref, b2m_ref, cntcol):
    i = pl.program_id(0)
    nb = pl.num_programs(0)
    x = x_ref[...]
    dom = dom_ref[0]
    k8 = jax.lax.broadcasted_iota(jnp.int32, (_KP, 1), 0)
    oh = (dom == k8).astype(jnp.float32)
    psum = jax.lax.dot_general(
        oh, x, (((1,), (0,)), ((), ())), preferred_element_type=jnp.float32)
    pcnt = jnp.sum(oh, axis=1, keepdims=True)

    @pl.when(i == 0)
    def _():
        cent_ref[...] = psum
        cntcol[...] = pcnt

    @pl.when(i > 0)
    def _():
        cent_ref[...] += psum
        cntcol[...] += pcnt

    @pl.when(i == nb - 1)
    def _():
        cnt = cntcol[...]
        cent = jnp.where(
            cnt > 0.0, cent_ref[...] / jnp.maximum(cnt, 1.0), 0.0)
        cent_ref[...] = cent
        cent3_ref[...] = _MIX * cent
        b2 = jnp.sum(cent * cent, axis=1, keepdims=True)
        b2m_ref[...] = jnp.where(cnt > 0.0, b2, _NEG)


def _apply_kernel(cent_ref, cent3_ref, b2m_ref, x_ref, out_ref):
    x = x_ref[...]
    cent = cent_ref[...]
    xcT = jax.lax.dot_general(
        cent, x, (((1,), (1,)), ((), ())),
        preferred_element_type=jnp.float32)
    score = b2m_ref[...] - 2.0 * xcT
    m4 = jnp.maximum(score[0:4], score[4:8])
    m2 = jnp.maximum(m4[0:2], m4[2:4])
    m1 = jnp.maximum(m2[0:1], m2[1:2])
    k8 = jax.lax.broadcasted_iota(jnp.int32, (_KP, 1), 0)
    kk = jnp.where(score == m1, k8, _KP)
    i4 = jnp.minimum(kk[0:4], kk[4:8])
    i2 = jnp.minimum(i4[0:2], i4[2:4])
    idx = jnp.minimum(i2[0:1], i2[1:2])
    ohsel = (k8 == idx).astype(jnp.float32)
    hardest3 = jax.lax.dot_general(
        ohsel, cent3_ref[...], (((0,), (0,)), ((), ())),
        preferred_element_type=jnp.float32)
    out_ref[...] = _KEEP * x + hardest3


@jax.jit
def kernel(c_vt, domains):
    dom3 = domains.reshape(_NB, 1, _R)
    cent, cent3, b2m = pl.pallas_call(
        _seg_kernel,
        grid=(_NB,),
        in_specs=[
            pl.BlockSpec((1, 1, _R), lambda i: (i, 0, 0)),
            pl.BlockSpec((_R, _D), lambda i: (i, 0)),
        ],
        out_specs=[
            pl.BlockSpec((_KP, _D), lambda i: (0, 0)),
            pl.BlockSpec((_KP, _D), lambda i: (0, 0)),
            pl.BlockSpec((_KP, 1), lambda i: (0, 0)),
        ],
        out_shape=[
            jax.ShapeDtypeStruct((_KP, _D), jnp.float32),
            jax.ShapeDtypeStruct((_KP, _D), jnp.float32),
            jax.ShapeDtypeStruct((_KP, 1), jnp.float32),
        ],
        scratch_shapes=[pltpu.VMEM((_KP, 1), jnp.float32)],
    )(dom3, c_vt)

    out = pl.pallas_call(
        _apply_kernel,
        grid=(_NB,),
        in_specs=[
            pl.BlockSpec((_KP, _D), lambda i: (0, 0)),
            pl.BlockSpec((_KP, _D), lambda i: (0, 0)),
            pl.BlockSpec((_KP, 1), lambda i: (0, 0)),
            pl.BlockSpec((_R, _D), lambda i: (i, 0)),
        ],
        out_specs=pl.BlockSpec((_R, _D), lambda i: (i, 0)),
        out_shape=jax.ShapeDtypeStruct((_B, _D), jnp.float32),
    )(cent, cent3, b2m, c_vt)
    return out


# speedup vs baseline: 10.0532x; 1.2767x over previous
import jax
import jax.numpy as jnp
from jax.experimental import pallas as pl
from jax.experimental.pallas import tpu as pltpu

_K = 7
_KP = 8
_D = 512
_B = 16384
_R = 2048
_NB = _B // _R
_MIX = 0.3
_KEEP = 1.0 - _MIX
_NEG = -1.0e30


def _fused_kernel(dom_ref, x_ref, out_ref, xbuf, cent_s, cent3_s, b2m_s,
                  cnt_s):
    p = pl.program_id(0)
    i = pl.program_id(1)
    nb = pl.num_programs(1)

    @pl.when(p == 0)
    def _phase0():
        x = x_ref[...]
        xbuf[pl.ds(i * _R, _R), :] = x
        dom = dom_ref[0]
        k8 = jax.lax.broadcasted_iota(jnp.int32, (_KP, 1), 0)
        oh = (dom == k8).astype(jnp.float32)
        psum = jax.lax.dot_general(
            oh, x, (((1,), (0,)), ((), ())),
            preferred_element_type=jnp.float32)
        pcnt = jnp.sum(oh, axis=1, keepdims=True)

        @pl.when(i == 0)
        def _():
            cent_s[...] = psum
            cnt_s[...] = pcnt

        @pl.when(i > 0)
        def _():
            cent_s[...] += psum
            cnt_s[...] += pcnt

        @pl.when(i == nb - 1)
        def _():
            cnt = cnt_s[...]
            cent = jnp.where(
                cnt > 0.0, cent_s[...] / jnp.maximum(cnt, 1.0), 0.0)
            cent_s[...] = cent
            cent3_s[...] = _MIX * cent
            b2 = jnp.sum(cent * cent, axis=1, keepdims=True)
            b2m_s[...] = jnp.where(cnt > 0.0, b2, _NEG)

    @pl.when(p == 1)
    def _phase1():
        x = xbuf[pl.ds(i * _R, _R), :]
        cent = cent_s[...]
        xcT = jax.lax.dot_general(
            cent, x, (((1,), (1,)), ((), ())),
            preferred_element_type=jnp.float32)
        score = b2m_s[...] - 2.0 * xcT
        m4 = jnp.maximum(score[0:4], score[4:8])
        m2 = jnp.maximum(m4[0:2], m4[2:4])
        m1 = jnp.maximum(m2[0:1], m2[1:2])
        k8 = jax.lax.broadcasted_iota(jnp.int32, (_KP, 1), 0)
        kk = jnp.where(score == m1, k8, _KP)
        i4 = jnp.minimum(kk[0:4], kk[4:8])
        i2 = jnp.minimum(i4[0:2], i4[2:4])
        idx = jnp.minimum(i2[0:1], i2[1:2])
        ohsel = (k8 == idx).astype(jnp.float32)
        hardest3 = jax.lax.dot_general(
            ohsel, cent3_s[...], (((0,), (0,)), ((), ())),
            preferred_element_type=jnp.float32)
        out_ref[...] = _KEEP * x + hardest3


@jax.jit
def kernel(c_vt, domains):
    dom3 = domains.reshape(_NB, 1, _R)
    out = pl.pallas_call(
        _fused_kernel,
        grid=(2, _NB),
        in_specs=[
            pl.BlockSpec((1, 1, _R), lambda p, i: (i * (1 - p), 0, 0)),
            pl.BlockSpec((_R, _D), lambda p, i: (i * (1 - p), 0)),
        ],
        out_specs=pl.BlockSpec((_R, _D), lambda p, i: (i * p, 0)),
        out_shape=jax.ShapeDtypeStruct((_B, _D), jnp.float32),
        scratch_shapes=[
            pltpu.VMEM((_B, _D), jnp.float32),
            pltpu.VMEM((_KP, _D), jnp.float32),
            pltpu.VMEM((_KP, _D), jnp.float32),
            pltpu.VMEM((_KP, 1), jnp.float32),
            pltpu.VMEM((_KP, 1), jnp.float32),
        ],
    )(dom3, c_vt)
    return out


# speedup vs baseline: 10.3697x; 1.0315x over previous
import jax
import jax.numpy as jnp
from jax.experimental import pallas as pl
from jax.experimental.pallas import tpu as pltpu

_K = 7
_KP = 8
_D = 512
_B = 16384
_R = 2048
_NB = _B // _R
_MIX = 0.3
_KEEP = 1.0 - _MIX
_NEG = -1.0e30


def _fused_kernel(dom_ref, x_ref, out_ref, xbuf, cent_s, cent3_s, b2m_s,
                  cnt_s):
    p = pl.program_id(0)
    i = pl.program_id(1)
    nb = pl.num_programs(1)

    @pl.when(p == 0)
    def _phase0():
        x = x_ref[...]
        xbuf[pl.ds(i * _R, _R), :] = x
        dom = dom_ref[0]
        k8 = jax.lax.broadcasted_iota(jnp.int32, (_KP, 1), 0)
        oh = (dom == k8).astype(jnp.float32)
        psum = jax.lax.dot_general(
            oh, x, (((1,), (0,)), ((), ())),
            preferred_element_type=jnp.float32)
        pcnt = jnp.sum(oh, axis=1, keepdims=True)

        @pl.when(i == 0)
        def _():
            cent_s[...] = psum
            cnt_s[...] = pcnt

        @pl.when(i > 0)
        def _():
            cent_s[...] += psum
            cnt_s[...] += pcnt

        @pl.when(i == nb - 1)
        def _():
            cnt = cnt_s[...]
            cent = jnp.where(
                cnt > 0.0, cent_s[...] / jnp.maximum(cnt, 1.0), 0.0)
            cent_s[...] = cent
            cent3_s[...] = _MIX * cent
            b2 = jnp.sum(cent * cent, axis=1, keepdims=True)
            b2m_s[...] = jnp.where(cnt > 0.0, b2, _NEG)

    @pl.when(p == 1)
    def _phase1():
        x = xbuf[pl.ds(i * _R, _R), :]
        cent = cent_s[...]
        xcT = jax.lax.dot_general(
            cent, x, (((1,), (1,)), ((), ())),
            preferred_element_type=jnp.float32)
        score = b2m_s[...] - 2.0 * xcT
        m4 = jnp.maximum(score[0:4], score[4:8])
        m2 = jnp.maximum(m4[0:2], m4[2:4])
        m1 = jnp.maximum(m2[0:1], m2[1:2])
        k8 = jax.lax.broadcasted_iota(jnp.int32, (_KP, 1), 0)
        kk = jnp.where(score == m1, k8, _KP)
        i4 = jnp.minimum(kk[0:4], kk[4:8])
        i2 = jnp.minimum(i4[0:2], i4[2:4])
        idx = jnp.minimum(i2[0:1], i2[1:2])
        ohsel = (k8 == idx).astype(jnp.float32)
        hardest3 = jax.lax.dot_general(
            ohsel, cent3_s[...], (((0,), (0,)), ((), ())),
            preferred_element_type=jnp.float32)
        out_ref[...] = _KEEP * x + hardest3


@jax.jit
def kernel(c_vt, domains):
    dom3 = domains.reshape(_NB, 1, _R)
    out = pl.pallas_call(
        _fused_kernel,
        grid=(2, _NB),
        in_specs=[
            pl.BlockSpec(
                (1, 1, _R),
                lambda p, i: (i * (1 - p) + (_NB - 1) * p, 0, 0)),
            pl.BlockSpec(
                (_R, _D),
                lambda p, i: (i * (1 - p) + (_NB - 1) * p, 0)),
        ],
        out_specs=pl.BlockSpec((_R, _D), lambda p, i: (i * p, 0)),
        out_shape=jax.ShapeDtypeStruct((_B, _D), jnp.float32),
        scratch_shapes=[
            pltpu.VMEM((_B, _D), jnp.float32),
            pltpu.VMEM((_KP, _D), jnp.float32),
            pltpu.VMEM((_KP, _D), jnp.float32),
            pltpu.VMEM((_KP, 1), jnp.float32),
            pltpu.VMEM((_KP, 1), jnp.float32),
        ],
    )(dom3, c_vt)
    return out
